# Initial kernel scaffold; baseline (speedup 1.0000x reference)
import functools

import jax
import jax.numpy as jnp
from jax import lax
from jax.experimental import pallas as pl
from jax.experimental.pallas import tpu as pltpu
from jax.experimental.pallas import tpu_sc as plsc

N = 10000
H = 128
E = 320000
NTILE = 32
EPT = E // 16
CW = 80
NCHUNK = EPT // CW
RPS = N // 16

ELP = 100352
SCW = 64
SCH = ELP // (32 * SCW)


def _sc_mesh():
    return plsc.VectorSubcoreMesh(core_axis_name="c", subcore_axis_name="s")


def _zero_rows(ref, nrows, ncolchunks):
    zf = jnp.zeros((16,), jnp.float32)

    def body(i, _):
        for j in range(ncolchunks):
            ref[i, pl.ds(j * 16, 16)] = zf
        return 0

    lax.fori_loop(0, nrows, body, 0)


def _agg_body(with_counts, msgs_hbm, src_hbm, dst_hbm, *refs):
    if with_counts:
        (S_out, C_out, idx_src, idx_dst, rb0, rb1, zrow, acc,
         ones, cnt, sem0, sem1) = refs
    else:
        (S_out, idx_src, idx_dst, rb0, rb1, zrow, acc, sem0, sem1) = refs

    c = lax.axis_index("c")
    s = lax.axis_index("s")
    w = c * 16 + s

    _zero_rows(zrow, 125, 8)
    for k in range(RPS // 125):
        pltpu.sync_copy(zrow, acc.at[pl.ds(s * RPS + k * 125, 125)])
    if with_counts:
        of = jnp.ones((16,), jnp.float32)

        def fill_ones(i, _):
            ones[i, :] = of
            return 0

        lax.fori_loop(0, CW, fill_ones, 0)
        for k in range(RPS // 125):
            pltpu.sync_copy(zrow.at[:, pl.ds(0, 16)],
                            cnt.at[pl.ds(s * RPS + k * 125, 125)])

    pltpu.sync_copy(src_hbm.at[w], idx_src)
    pltpu.sync_copy(dst_hbm.at[w], idx_dst)

    offv = jnp.zeros((16,), jnp.int32) + c * N

    def offset_body(i, _):
        for j in range(CW // 16):
            sl = pl.ds(j * 16, 16)
            idx_src[i, sl] = idx_src[i, sl] + offv
        return 0

    lax.fori_loop(0, NCHUNK, offset_body, 0)

    plsc.subcore_barrier()

    def chunk_body(i, _):
        c0 = 2 * i
        d0 = pltpu.async_copy(msgs_hbm.at[idx_src.at[c0]], rb0, sem0)
        d1 = pltpu.async_copy(msgs_hbm.at[idx_src.at[c0 + 1]], rb1, sem1)
        d0.wait()
        pltpu.sync_copy(rb0, acc.at[idx_dst.at[c0]], add=True)
        if with_counts:
            pltpu.sync_copy(ones, cnt.at[idx_dst.at[c0]], add=True)
        d1.wait()
        pltpu.sync_copy(rb1, acc.at[idx_dst.at[c0 + 1]], add=True)
        if with_counts:
            pltpu.sync_copy(ones, cnt.at[idx_dst.at[c0 + 1]], add=True)
        return 0

    lax.fori_loop(0, NCHUNK // 2, chunk_body, 0)

    plsc.subcore_barrier()

    row0 = c * N + s * RPS
    pltpu.sync_copy(acc.at[pl.ds(s * RPS, RPS)], S_out.at[pl.ds(row0, RPS)])
    if with_counts:
        pltpu.sync_copy(cnt.at[pl.ds(s * RPS, RPS)], C_out.at[pl.ds(row0, RPS)])


def _sc_aggregate(msgs, src3, dst3, with_counts):
    out_type = [jax.ShapeDtypeStruct((2 * N, H), jnp.float32)]
    scratch = [
        pltpu.VMEM((NCHUNK, CW), jnp.int32),
        pltpu.VMEM((NCHUNK, CW), jnp.int32),
        pltpu.VMEM((CW, H), jnp.float32),
        pltpu.VMEM((CW, H), jnp.float32),
        pltpu.VMEM((125, H), jnp.float32),
        pltpu.VMEM_SHARED((N, H), jnp.float32),
    ]
    if with_counts:
        out_type.append(jax.ShapeDtypeStruct((2 * N, 16), jnp.float32))
        scratch += [
            pltpu.VMEM((CW, 16), jnp.float32),
            pltpu.VMEM_SHARED((N, 16), jnp.float32),
        ]
    scratch += [pltpu.SemaphoreType.DMA, pltpu.SemaphoreType.DMA]
    return pl.kernel(
        functools.partial(_agg_body, with_counts),
        out_type=out_type,
        mesh=_sc_mesh(),
        scratch_types=scratch,
    )(msgs, src3, dst3)


def _score_body(hu_hbm, hr_hbm, el0_hbm, el1_hbm, out_hbm,
                idx0, idx1, ub, rb, sc, sem0, sem1):
    c = lax.axis_index("c")
    s = lax.axis_index("s")
    w = c * 16 + s
    pltpu.sync_copy(el0_hbm.at[pl.ds(w * SCH, SCH)], idx0)
    pltpu.sync_copy(el1_hbm.at[pl.ds(w * SCH, SCH)], idx1)

    def chunk(i, _):
        d0 = pltpu.async_copy(hu_hbm.at[idx0.at[i]], ub, sem0)
        d1 = pltpu.async_copy(hr_hbm.at[idx1.at[i]], rb, sem1)
        d0.wait()
        d1.wait()
        for g in range(SCW // 16):
            rows = lax.iota(jnp.int32, 16) + g * 16

            def hbody(h, a):
                cols = jnp.zeros((16,), jnp.int32) + h
                u = plsc.load_gather(ub, [rows, cols])
                r = plsc.load_gather(rb, [rows, cols])
                return a + u * r

            a = lax.fori_loop(0, H, hbody, jnp.zeros((16,), jnp.float32))
            sc[i, pl.ds(g * 16, 16)] = a
        return 0

    lax.fori_loop(0, SCH, chunk, 0)
    pltpu.sync_copy(sc, out_hbm.at[pl.ds(w * SCH, SCH)])


def _sc_score(hu2, hr2, el0, el1):
    return pl.kernel(
        _score_body,
        out_type=jax.ShapeDtypeStruct((ELP // SCW, SCW), jnp.float32),
        mesh=_sc_mesh(),
        scratch_types=[
            pltpu.VMEM((SCH, SCW), jnp.int32),
            pltpu.VMEM((SCH, SCW), jnp.int32),
            pltpu.VMEM((SCW, H), jnp.float32),
            pltpu.VMEM((SCW, H), jnp.float32),
            pltpu.VMEM((SCH, SCW), jnp.float32),
            pltpu.SemaphoreType.DMA,
            pltpu.SemaphoreType.DMA,
        ],
    )(hu2, hr2, el0, el1)


_ROWS = 1000


def _tc_encode(x, emb, W, b, Wl, Wr):
    n, k = x.shape

    def body(x_ref, emb_ref, W_ref, b_ref, Wl_ref, Wr_ref, p_ref, s_ref):
        h = jnp.dot(x_ref[...], W_ref[...], preferred_element_type=jnp.float32)
        h = h + b_ref[...] + emb_ref[...]
        p_ref[...] = jnp.dot(h, Wl_ref[...], preferred_element_type=jnp.float32)
        s_ref[...] = jnp.dot(h, Wr_ref[...], preferred_element_type=jnp.float32)

    return pl.pallas_call(
        body,
        grid=(n // _ROWS,),
        in_specs=[
            pl.BlockSpec((_ROWS, k), lambda i: (i, 0)),
            pl.BlockSpec((_ROWS, H), lambda i: (i, 0)),
            pl.BlockSpec((k, H), lambda i: (0, 0)),
            pl.BlockSpec((1, H), lambda i: (0, 0)),
            pl.BlockSpec((H, H), lambda i: (0, 0)),
            pl.BlockSpec((H, H), lambda i: (0, 0)),
        ],
        out_specs=[pl.BlockSpec((_ROWS, H), lambda i: (i, 0))] * 2,
        out_shape=[jax.ShapeDtypeStruct((n, H), jnp.float32)] * 2,
    )(x, emb, W, b.reshape(1, H), Wl, Wr)


def _tc_layer_mid(S, C, bl, st, Wl2, Wr2):
    def body(S_ref, C_ref, bl_ref, st_ref, Wl_ref, Wr_ref, p_ref, s_ref):
        inv = 1.0 / jnp.maximum(C_ref[...][:, 0:1], 1.0)
        h = jnp.maximum(S_ref[...] * inv + bl_ref[...] + st_ref[...], 0.0)
        p_ref[...] = jnp.dot(h, Wl_ref[...], preferred_element_type=jnp.float32)
        s_ref[...] = jnp.dot(h, Wr_ref[...], preferred_element_type=jnp.float32)

    return pl.pallas_call(
        body,
        grid=(N // _ROWS,),
        in_specs=[
            pl.BlockSpec((_ROWS, H), lambda i: (i, 0)),
            pl.BlockSpec((_ROWS, 16), lambda i: (i, 0)),
            pl.BlockSpec((1, H), lambda i: (0, 0)),
            pl.BlockSpec((_ROWS, H), lambda i: (i, 0)),
            pl.BlockSpec((H, H), lambda i: (0, 0)),
            pl.BlockSpec((H, H), lambda i: (0, 0)),
        ],
        out_specs=[pl.BlockSpec((_ROWS, H), lambda i: (i, 0))] * 2,
        out_shape=[jax.ShapeDtypeStruct((N, H), jnp.float32)] * 2,
    )(S, C, bl.reshape(1, H), st, Wl2, Wr2)


def _tc_final(S2, C, bl2, st):
    def body(S_ref, C_ref, bl_ref, st_ref, o_ref):
        inv = 1.0 / jnp.maximum(C_ref[...][:, 0:1], 1.0)
        o_ref[...] = S_ref[...] * inv + bl_ref[...] + st_ref[...]

    return pl.pallas_call(
        body,
        grid=(N // _ROWS,),
        in_specs=[
            pl.BlockSpec((_ROWS, H), lambda i: (i, 0)),
            pl.BlockSpec((_ROWS, 16), lambda i: (i, 0)),
            pl.BlockSpec((1, H), lambda i: (0, 0)),
            pl.BlockSpec((_ROWS, H), lambda i: (i, 0)),
        ],
        out_specs=pl.BlockSpec((_ROWS, H), lambda i: (i, 0)),
        out_shape=jax.ShapeDtypeStruct((N, H), jnp.float32),
    )(S2, C, bl2.reshape(1, H), st)


def kernel(x_user, x_recipe, node_id_user, node_id_recipe, edge_index_u2r,
           edge_index_r2u, edge_label_index, W_user_lin, b_user_lin,
           W_recipe_lin, b_recipe_lin, emb_user, emb_recipe,
           Wl1_u2r, bl1_u2r, Wr1_u2r, Wl1_r2u, bl1_r2u, Wr1_r2u,
           Wl2_u2r, bl2_u2r, Wr2_u2r, Wl2_r2u, bl2_r2u, Wr2_r2u):
    xup = jnp.pad(x_user, ((0, 0), (0, 6)))
    Wup = jnp.pad(W_user_lin, ((0, 6), (0, 0)))

    pu1, su1 = _tc_encode(xup, emb_user, Wup, b_user_lin, Wl1_u2r, Wr1_r2u)
    pr1, sr1 = _tc_encode(x_recipe, emb_recipe, W_recipe_lin, b_recipe_lin,
                          Wl1_r2u, Wr1_u2r)

    msgs1 = jnp.concatenate([pr1, pu1], axis=0)
    src_all = jnp.concatenate(
        [edge_index_r2u[0], edge_index_u2r[0]]).reshape(NTILE, NCHUNK, CW)
    dst_all = jnp.concatenate(
        [edge_index_r2u[1], edge_index_u2r[1]]).reshape(NTILE, NCHUNK, CW)

    S1, C = _sc_aggregate(msgs1, src_all, dst_all, with_counts=True)
    Cu, Cr = C[:N], C[N:]

    pu2, su2 = _tc_layer_mid(S1[:N], Cu, bl1_r2u, su1, Wl2_u2r, Wr2_r2u)
    pr2, sr2 = _tc_layer_mid(S1[N:], Cr, bl1_u2r, sr1, Wl2_r2u, Wr2_u2r)

    msgs2 = jnp.concatenate([pr2, pu2], axis=0)
    (S2,) = _sc_aggregate(msgs2, src_all, dst_all, with_counts=False)

    hu2 = _tc_final(S2[:N], Cu, bl2_r2u, su2)
    hr2 = _tc_final(S2[N:], Cr, bl2_u2r, sr2)

    pad = jnp.zeros((ELP - edge_label_index.shape[1],), jnp.int32)
    el0 = jnp.concatenate([edge_label_index[0], pad]).reshape(ELP // SCW, SCW)
    el1 = jnp.concatenate([edge_label_index[1], pad]).reshape(ELP // SCW, SCW)
    scores = _sc_score(hu2, hr2, el0, el1)
    return scores.reshape(-1)[:edge_label_index.shape[1]]


# speedup vs baseline: 3.7906x; 3.7906x over previous
import functools

import jax
import jax.numpy as jnp
from jax import lax
from jax.experimental import pallas as pl
from jax.experimental.pallas import tpu as pltpu
from jax.experimental.pallas import tpu_sc as plsc

N = 10000
NP = 10240
H = 128
HH = H // 2
E = 320000
NTILE = 32
EPT = E // 16
CW = 80
NCHUNK = EPT // CW
RPS = NP // 16

ELP = 100352
SCW = 64
SCH = ELP // (32 * SCW)


def _sc_mesh():
    return plsc.VectorSubcoreMesh(core_axis_name="c", subcore_axis_name="s")


def _zero_rows(ref, nrows, ncolchunks):
    zf = jnp.zeros((16,), jnp.float32)

    def body(i, _):
        for j in range(ncolchunks):
            ref[i, pl.ds(j * 16, 16)] = zf
        return 0

    lax.fori_loop(0, nrows, body, 0)


def _agg_body(with_counts, msgs0_hbm, msgs1_hbm, src_hbm, dst_hbm, *refs):
    if with_counts:
        (S0_out, S1_out, C_out, idx_src, idx_dst, rb0, rb1, zrow, acc,
         ones, zcnt, cnt, sem0, sem1) = refs
    else:
        (S0_out, S1_out, idx_src, idx_dst, rb0, rb1, zrow, acc,
         sem0, sem1) = refs

    c = lax.axis_index("c")
    s = lax.axis_index("s")
    w = c * 16 + s

    _zero_rows(zrow, 128, HH // 16)
    if with_counts:
        of = jnp.ones((16,), jnp.float32)

        def fill_ones(i, _):
            ones[i, :] = of
            return 0

        lax.fori_loop(0, CW, fill_ones, 0)
        _zero_rows(zcnt, 128, 1)

    pltpu.sync_copy(src_hbm.at[w], idx_src)
    pltpu.sync_copy(dst_hbm.at[w], idx_dst)

    offv = jnp.zeros((16,), jnp.int32) + c * N

    def offset_body(i, _):
        for j in range(CW // 16):
            sl = pl.ds(j * 16, 16)
            idx_src[i, sl] = idx_src[i, sl] + offv
        return 0

    lax.fori_loop(0, NCHUNK, offset_body, 0)

    for half in range(2):
        msgs_hbm = (msgs0_hbm, msgs1_hbm)[half]
        S_out = (S0_out, S1_out)[half]
        counts = with_counts and half == 0

        for k in range(RPS // 128):
            pltpu.sync_copy(zrow, acc.at[pl.ds(s * RPS + k * 128, 128)])
        if counts:
            for k in range(RPS // 128):
                pltpu.sync_copy(zcnt, cnt.at[pl.ds(s * RPS + k * 128, 128)])

        plsc.subcore_barrier()

        def chunk_body(i, _):
            c0 = 2 * i
            d0 = pltpu.async_copy(msgs_hbm.at[idx_src.at[c0]], rb0, sem0)
            d1 = pltpu.async_copy(msgs_hbm.at[idx_src.at[c0 + 1]], rb1, sem1)
            d0.wait()
            pltpu.sync_copy(rb0, acc.at[idx_dst.at[c0]], add=True)
            if counts:
                pltpu.sync_copy(ones, cnt.at[idx_dst.at[c0]], add=True)
            d1.wait()
            pltpu.sync_copy(rb1, acc.at[idx_dst.at[c0 + 1]], add=True)
            if counts:
                pltpu.sync_copy(ones, cnt.at[idx_dst.at[c0 + 1]], add=True)
            return 0

        lax.fori_loop(0, NCHUNK // 2, chunk_body, 0)

        plsc.subcore_barrier()

        pltpu.sync_copy(acc.at[pl.ds(s * RPS, RPS)], S_out.at[w])
        if counts:
            pltpu.sync_copy(cnt.at[pl.ds(s * RPS, RPS)], C_out.at[w])


def _sc_aggregate(msgs0, msgs1, src3, dst3, with_counts):
    out_type = [jax.ShapeDtypeStruct((NTILE, RPS, HH), jnp.float32)] * 2
    scratch = [
        pltpu.VMEM((NCHUNK, CW), jnp.int32),
        pltpu.VMEM((NCHUNK, CW), jnp.int32),
        pltpu.VMEM((CW, HH), jnp.float32),
        pltpu.VMEM((CW, HH), jnp.float32),
        pltpu.VMEM((128, HH), jnp.float32),
        pltpu.VMEM_SHARED((NP, HH), jnp.float32),
    ]
    if with_counts:
        out_type.append(jax.ShapeDtypeStruct((NTILE, RPS, 16), jnp.float32))
        scratch += [
            pltpu.VMEM((CW, 16), jnp.float32),
            pltpu.VMEM((128, 16), jnp.float32),
            pltpu.VMEM_SHARED((NP, 16), jnp.float32),
        ]
    scratch += [pltpu.SemaphoreType.DMA, pltpu.SemaphoreType.DMA]
    return pl.kernel(
        functools.partial(_agg_body, with_counts),
        out_type=out_type,
        mesh=_sc_mesh(),
        scratch_types=scratch,
        compiler_params=pltpu.CompilerParams(use_tc_tiling_on_sc=False),
    )(msgs0, msgs1, src3, dst3)


def _score_body(hu_hbm, hr_hbm, el0_hbm, el1_hbm, out_hbm,
                idx0, idx1, ub, rb, sc, sem0, sem1):
    c = lax.axis_index("c")
    s = lax.axis_index("s")
    w = c * 16 + s
    pltpu.sync_copy(el0_hbm.at[w], idx0)
    pltpu.sync_copy(el1_hbm.at[w], idx1)

    def chunk(i, _):
        d0 = pltpu.async_copy(hu_hbm.at[idx0.at[i]], ub, sem0)
        d1 = pltpu.async_copy(hr_hbm.at[idx1.at[i]], rb, sem1)
        d0.wait()
        d1.wait()
        for g in range(SCW // 16):
            rows = lax.iota(jnp.int32, 16) + g * 16

            def hbody(h, a):
                cols = jnp.zeros((16,), jnp.int32) + h
                u = plsc.load_gather(ub, [rows, cols])
                r = plsc.load_gather(rb, [rows, cols])
                return a + u * r

            a = lax.fori_loop(0, H, hbody, jnp.zeros((16,), jnp.float32))
            sc[i, pl.ds(g * 16, 16)] = a
        return 0

    lax.fori_loop(0, SCH, chunk, 0)
    pltpu.sync_copy(sc, out_hbm.at[w])


def _sc_score(hu2, hr2, el0, el1):
    return pl.kernel(
        _score_body,
        out_type=jax.ShapeDtypeStruct((NTILE, SCH, SCW), jnp.float32),
        mesh=_sc_mesh(),
        scratch_types=[
            pltpu.VMEM((SCH, SCW), jnp.int32),
            pltpu.VMEM((SCH, SCW), jnp.int32),
            pltpu.VMEM((SCW, H), jnp.float32),
            pltpu.VMEM((SCW, H), jnp.float32),
            pltpu.VMEM((SCH, SCW), jnp.float32),
            pltpu.SemaphoreType.DMA,
            pltpu.SemaphoreType.DMA,
        ],
        compiler_params=pltpu.CompilerParams(needs_layout_passes=False),
    )(hu2, hr2, el0, el1)


_ROWS = 1000


def _tc_encode(x, emb, W, b, Wl, Wr):
    n, k = x.shape

    def body(x_ref, emb_ref, W_ref, b_ref, Wl_ref, Wr_ref, p_ref, s_ref):
        h = jnp.dot(x_ref[...], W_ref[...], preferred_element_type=jnp.float32)
        h = h + b_ref[...] + emb_ref[...]
        p_ref[...] = jnp.dot(h, Wl_ref[...], preferred_element_type=jnp.float32)
        s_ref[...] = jnp.dot(h, Wr_ref[...], preferred_element_type=jnp.float32)

    return pl.pallas_call(
        body,
        grid=(n // _ROWS,),
        in_specs=[
            pl.BlockSpec((_ROWS, k), lambda i: (i, 0)),
            pl.BlockSpec((_ROWS, H), lambda i: (i, 0)),
            pl.BlockSpec((k, H), lambda i: (0, 0)),
            pl.BlockSpec((1, H), lambda i: (0, 0)),
            pl.BlockSpec((H, H), lambda i: (0, 0)),
            pl.BlockSpec((H, H), lambda i: (0, 0)),
        ],
        out_specs=[pl.BlockSpec((_ROWS, H), lambda i: (i, 0))] * 2,
        out_shape=[jax.ShapeDtypeStruct((n, H), jnp.float32)] * 2,
    )(x, emb, W, b.reshape(1, H), Wl, Wr)


def _tc_layer_mid(S, C, bl, st, Wl2, Wr2):
    def body(S_ref, C_ref, bl_ref, st_ref, Wl_ref, Wr_ref, p_ref, s_ref):
        inv = 1.0 / jnp.maximum(C_ref[...][:, 0:1], 1.0)
        h = jnp.maximum(S_ref[...] * inv + bl_ref[...] + st_ref[...], 0.0)
        p_ref[...] = jnp.dot(h, Wl_ref[...], preferred_element_type=jnp.float32)
        s_ref[...] = jnp.dot(h, Wr_ref[...], preferred_element_type=jnp.float32)

    return pl.pallas_call(
        body,
        grid=(N // _ROWS,),
        in_specs=[
            pl.BlockSpec((_ROWS, H), lambda i: (i, 0)),
            pl.BlockSpec((_ROWS, 16), lambda i: (i, 0)),
            pl.BlockSpec((1, H), lambda i: (0, 0)),
            pl.BlockSpec((_ROWS, H), lambda i: (i, 0)),
            pl.BlockSpec((H, H), lambda i: (0, 0)),
            pl.BlockSpec((H, H), lambda i: (0, 0)),
        ],
        out_specs=[pl.BlockSpec((_ROWS, H), lambda i: (i, 0))] * 2,
        out_shape=[jax.ShapeDtypeStruct((N, H), jnp.float32)] * 2,
    )(S, C, bl.reshape(1, H), st, Wl2, Wr2)


def _tc_final(S2, C, bl2, st):
    def body(S_ref, C_ref, bl_ref, st_ref, o_ref):
        inv = 1.0 / jnp.maximum(C_ref[...][:, 0:1], 1.0)
        o_ref[...] = S_ref[...] * inv + bl_ref[...] + st_ref[...]

    return pl.pallas_call(
        body,
        grid=(N // _ROWS,),
        in_specs=[
            pl.BlockSpec((_ROWS, H), lambda i: (i, 0)),
            pl.BlockSpec((_ROWS, 16), lambda i: (i, 0)),
            pl.BlockSpec((1, H), lambda i: (0, 0)),
            pl.BlockSpec((_ROWS, H), lambda i: (i, 0)),
        ],
        out_specs=pl.BlockSpec((_ROWS, H), lambda i: (i, 0)),
        out_shape=jax.ShapeDtypeStruct((N, H), jnp.float32),
    )(S2, C, bl2.reshape(1, H), st)


def kernel(x_user, x_recipe, node_id_user, node_id_recipe, edge_index_u2r,
           edge_index_r2u, edge_label_index, W_user_lin, b_user_lin,
           W_recipe_lin, b_recipe_lin, emb_user, emb_recipe,
           Wl1_u2r, bl1_u2r, Wr1_u2r, Wl1_r2u, bl1_r2u, Wr1_r2u,
           Wl2_u2r, bl2_u2r, Wr2_u2r, Wl2_r2u, bl2_r2u, Wr2_r2u):
    xup = jnp.pad(x_user, ((0, 0), (0, 6)))
    Wup = jnp.pad(W_user_lin, ((0, 6), (0, 0)))

    pu1, su1 = _tc_encode(xup, emb_user, Wup, b_user_lin, Wl1_u2r, Wr1_r2u)
    pr1, sr1 = _tc_encode(x_recipe, emb_recipe, W_recipe_lin, b_recipe_lin,
                          Wl1_r2u, Wr1_u2r)

    msgs1 = jnp.concatenate([pr1, pu1], axis=0)
    src_all = jnp.concatenate(
        [edge_index_r2u[0], edge_index_u2r[0]]).reshape(NTILE, NCHUNK, CW)
    dst_all = jnp.concatenate(
        [edge_index_r2u[1], edge_index_u2r[1]]).reshape(NTILE, NCHUNK, CW)

    S1a, S1b, C = _sc_aggregate(msgs1[:, :HH], msgs1[:, HH:],
                                src_all, dst_all, with_counts=True)
    S1 = jnp.concatenate(
        [S1a.reshape(2, NP, HH), S1b.reshape(2, NP, HH)], axis=-1)
    C = C.reshape(2, NP, 16)
    Cu, Cr = C[0, :N], C[1, :N]

    pu2, su2 = _tc_layer_mid(S1[0, :N], Cu, bl1_r2u, su1, Wl2_u2r, Wr2_r2u)
    pr2, sr2 = _tc_layer_mid(S1[1, :N], Cr, bl1_u2r, sr1, Wl2_r2u, Wr2_u2r)

    msgs2 = jnp.concatenate([pr2, pu2], axis=0)
    S2a, S2b = _sc_aggregate(msgs2[:, :HH], msgs2[:, HH:],
                             src_all, dst_all, with_counts=False)
    S2 = jnp.concatenate(
        [S2a.reshape(2, NP, HH), S2b.reshape(2, NP, HH)], axis=-1)

    hu2 = _tc_final(S2[0, :N], Cu, bl2_r2u, su2)
    hr2 = _tc_final(S2[1, :N], Cr, bl2_u2r, sr2)

    pad = jnp.zeros((ELP - edge_label_index.shape[1],), jnp.int32)
    el0 = jnp.concatenate([edge_label_index[0], pad]).reshape(NTILE, SCH, SCW)
    el1 = jnp.concatenate([edge_label_index[1], pad]).reshape(NTILE, SCH, SCW)
    scores = _sc_score(hu2, hr2, el0, el1)
    return scores.reshape(-1)[:edge_label_index.shape[1]]


# speedup vs baseline: 5.4569x; 1.4396x over previous
import functools

import jax
import jax.numpy as jnp
from jax import lax
from jax.experimental import pallas as pl
from jax.experimental.pallas import tpu as pltpu
from jax.experimental.pallas import tpu_sc as plsc

N = 10000
NP = 10240
H = 128
HH = H // 2
E = 320000
NTILE = 32
EPT = E // 16
CW = 80
NCHUNK = EPT // CW
NB = 5
RPS = NP // 16

ELP = 102400
SCW = 64
SCH = ELP // (32 * SCW)


def _sc_mesh():
    return plsc.VectorSubcoreMesh(core_axis_name="c", subcore_axis_name="s")


def _zero_rows(ref, nrows, ncolchunks):
    zf = jnp.zeros((16,), jnp.float32)

    def body(i, _):
        for j in range(ncolchunks):
            ref[i, pl.ds(j * 16, 16)] = zf
        return 0

    lax.fori_loop(0, nrows, body, 0)


def _agg_body(with_counts, msgs0_hbm, msgs1_hbm, src_hbm, dst_hbm, *refs):
    if with_counts:
        (S0_out, S1_out, C_out, idx_src, idx_dst, zrow, acc,
         ones, zcnt, cnt) = refs[:10]
        rbs = refs[10:10 + NB]
        sems = refs[10 + NB:]
    else:
        (S0_out, S1_out, idx_src, idx_dst, zrow, acc) = refs[:6]
        rbs = refs[6:6 + NB]
        sems = refs[6 + NB:]

    c = lax.axis_index("c")
    s = lax.axis_index("s")
    w = c * 16 + s

    _zero_rows(zrow, 128, HH // 16)
    if with_counts:
        of = jnp.ones((16,), jnp.float32)

        def fill_ones(i, _):
            ones[i, :] = of
            return 0

        lax.fori_loop(0, CW, fill_ones, 0)
        _zero_rows(zcnt, 128, 1)

    pltpu.sync_copy(src_hbm.at[w], idx_src)
    pltpu.sync_copy(dst_hbm.at[w], idx_dst)

    offv = jnp.zeros((16,), jnp.int32) + c * N

    def offset_body(i, _):
        for j in range(CW // 16):
            sl = pl.ds(j * 16, 16)
            idx_src[i, sl] = idx_src[i, sl] + offv
        return 0

    lax.fori_loop(0, NCHUNK, offset_body, 0)

    for half in range(2):
        msgs_hbm = (msgs0_hbm, msgs1_hbm)[half]
        S_out = (S0_out, S1_out)[half]
        counts = with_counts and half == 0

        for k in range(RPS // 128):
            pltpu.sync_copy(zrow, acc.at[pl.ds(s * RPS + k * 128, 128)])
        if counts:
            for k in range(RPS // 128):
                pltpu.sync_copy(zcnt, cnt.at[pl.ds(s * RPS + k * 128, 128)])

        plsc.subcore_barrier()

        for b in range(NB):
            pltpu.async_copy(msgs_hbm.at[idx_src.at[b]], rbs[b], sems[b])

        def chunk_body(i, _):
            for b in range(NB):
                cch = i * NB + b
                pltpu.make_async_copy(
                    msgs_hbm.at[idx_src.at[cch]], rbs[b], sems[b]).wait()
                pltpu.sync_copy(rbs[b], acc.at[idx_dst.at[cch]], add=True)
                if counts:
                    pltpu.sync_copy(ones, cnt.at[idx_dst.at[cch]], add=True)

                @pl.when(cch + NB < NCHUNK)
                def _():
                    pltpu.async_copy(
                        msgs_hbm.at[idx_src.at[cch + NB]], rbs[b], sems[b])
            return 0

        lax.fori_loop(0, NCHUNK // NB, chunk_body, 0)

        plsc.subcore_barrier()

        pltpu.sync_copy(acc.at[pl.ds(s * RPS, RPS)], S_out.at[w])
        if counts:
            pltpu.sync_copy(cnt.at[pl.ds(s * RPS, RPS)], C_out.at[w])


def _sc_aggregate(msgs0, msgs1, src3, dst3, with_counts):
    out_type = [jax.ShapeDtypeStruct((NTILE, RPS, HH), jnp.float32)] * 2
    scratch = [
        pltpu.VMEM((NCHUNK, CW), jnp.int32),
        pltpu.VMEM((NCHUNK, CW), jnp.int32),
        pltpu.VMEM((128, HH), jnp.float32),
        pltpu.VMEM_SHARED((NP, HH), jnp.float32),
    ]
    if with_counts:
        out_type.append(jax.ShapeDtypeStruct((NTILE, RPS, 16), jnp.float32))
        scratch += [
            pltpu.VMEM((CW, 16), jnp.float32),
            pltpu.VMEM((128, 16), jnp.float32),
            pltpu.VMEM_SHARED((NP, 16), jnp.float32),
        ]
    scratch += [pltpu.VMEM((CW, HH), jnp.float32)] * NB
    scratch += [pltpu.SemaphoreType.DMA] * NB
    return pl.kernel(
        functools.partial(_agg_body, with_counts),
        out_type=out_type,
        mesh=_sc_mesh(),
        scratch_types=scratch,
        compiler_params=pltpu.CompilerParams(use_tc_tiling_on_sc=False),
    )(msgs0, msgs1, src3, dst3)


def _score_body(hu_hbm, hr_hbm, el0_hbm, el1_hbm, out_hbm,
                idx0, idx1, ub0, ub1, rb0, rb1, sc,
                usem0, usem1, rsem0, rsem1):
    c = lax.axis_index("c")
    s = lax.axis_index("s")
    w = c * 16 + s
    pltpu.sync_copy(el0_hbm.at[w], idx0)
    pltpu.sync_copy(el1_hbm.at[w], idx1)

    ubs, rbs = (ub0, ub1), (rb0, rb1)
    usems, rsems = (usem0, usem1), (rsem0, rsem1)
    for b in range(2):
        pltpu.async_copy(hu_hbm.at[idx0.at[b]], ubs[b], usems[b])
        pltpu.async_copy(hr_hbm.at[idx1.at[b]], rbs[b], rsems[b])

    def chunk(i, _):
        for b in range(2):
            cch = 2 * i + b
            ub, rb = ubs[b], rbs[b]
            pltpu.make_async_copy(hu_hbm.at[idx0.at[cch]], ub,
                                  usems[b]).wait()
            pltpu.make_async_copy(hr_hbm.at[idx1.at[cch]], rb,
                                  rsems[b]).wait()
            for g in range(SCW // 16):
                rows = lax.iota(jnp.int32, 16) + g * 16
                a = jnp.zeros((16,), jnp.float32)

                def hblk(ho, a):
                    for hh in range(8):
                        cols = jnp.zeros((16,), jnp.int32) + (ho * 8 + hh)
                        u = plsc.load_gather(ub, [rows, cols])
                        r = plsc.load_gather(rb, [rows, cols])
                        a = a + u * r
                    return a

                a = lax.fori_loop(0, H // 8, hblk, a)
                sc[cch, pl.ds(g * 16, 16)] = a

            @pl.when(cch + 2 < SCH)
            def _():
                pltpu.async_copy(hu_hbm.at[idx0.at[cch + 2]], ub, usems[b])
                pltpu.async_copy(hr_hbm.at[idx1.at[cch + 2]], rb, rsems[b])
        return 0

    lax.fori_loop(0, SCH // 2, chunk, 0)
    pltpu.sync_copy(sc, out_hbm.at[w])


def _sc_score(hu2, hr2, el0, el1):
    return pl.kernel(
        _score_body,
        out_type=jax.ShapeDtypeStruct((NTILE, SCH, SCW), jnp.float32),
        mesh=_sc_mesh(),
        scratch_types=[
            pltpu.VMEM((SCH, SCW), jnp.int32),
            pltpu.VMEM((SCH, SCW), jnp.int32),
            pltpu.VMEM((SCW, H), jnp.float32),
            pltpu.VMEM((SCW, H), jnp.float32),
            pltpu.VMEM((SCW, H), jnp.float32),
            pltpu.VMEM((SCW, H), jnp.float32),
            pltpu.VMEM((SCH, SCW), jnp.float32),
            pltpu.SemaphoreType.DMA,
            pltpu.SemaphoreType.DMA,
            pltpu.SemaphoreType.DMA,
            pltpu.SemaphoreType.DMA,
        ],
        compiler_params=pltpu.CompilerParams(needs_layout_passes=False),
    )(hu2, hr2, el0, el1)


_ROWS = 1000


def _tc_encode(x, emb, W, b, Wl, Wr):
    n, k = x.shape

    def body(x_ref, emb_ref, W_ref, b_ref, Wl_ref, Wr_ref, p_ref, s_ref):
        h = jnp.dot(x_ref[...], W_ref[...], preferred_element_type=jnp.float32)
        h = h + b_ref[...] + emb_ref[...]
        p_ref[...] = jnp.dot(h, Wl_ref[...], preferred_element_type=jnp.float32)
        s_ref[...] = jnp.dot(h, Wr_ref[...], preferred_element_type=jnp.float32)

    return pl.pallas_call(
        body,
        grid=(n // _ROWS,),
        in_specs=[
            pl.BlockSpec((_ROWS, k), lambda i: (i, 0)),
            pl.BlockSpec((_ROWS, H), lambda i: (i, 0)),
            pl.BlockSpec((k, H), lambda i: (0, 0)),
            pl.BlockSpec((1, H), lambda i: (0, 0)),
            pl.BlockSpec((H, H), lambda i: (0, 0)),
            pl.BlockSpec((H, H), lambda i: (0, 0)),
        ],
        out_specs=[pl.BlockSpec((_ROWS, H), lambda i: (i, 0))] * 2,
        out_shape=[jax.ShapeDtypeStruct((n, H), jnp.float32)] * 2,
    )(x, emb, W, b.reshape(1, H), Wl, Wr)


def _tc_layer_mid(S, C, bl, st, Wl2, Wr2):
    def body(S_ref, C_ref, bl_ref, st_ref, Wl_ref, Wr_ref, p_ref, s_ref):
        inv = 1.0 / jnp.maximum(C_ref[...][:, 0:1], 1.0)
        h = jnp.maximum(S_ref[...] * inv + bl_ref[...] + st_ref[...], 0.0)
        p_ref[...] = jnp.dot(h, Wl_ref[...], preferred_element_type=jnp.float32)
        s_ref[...] = jnp.dot(h, Wr_ref[...], preferred_element_type=jnp.float32)

    return pl.pallas_call(
        body,
        grid=(N // _ROWS,),
        in_specs=[
            pl.BlockSpec((_ROWS, H), lambda i: (i, 0)),
            pl.BlockSpec((_ROWS, 16), lambda i: (i, 0)),
            pl.BlockSpec((1, H), lambda i: (0, 0)),
            pl.BlockSpec((_ROWS, H), lambda i: (i, 0)),
            pl.BlockSpec((H, H), lambda i: (0, 0)),
            pl.BlockSpec((H, H), lambda i: (0, 0)),
        ],
        out_specs=[pl.BlockSpec((_ROWS, H), lambda i: (i, 0))] * 2,
        out_shape=[jax.ShapeDtypeStruct((N, H), jnp.float32)] * 2,
    )(S, C, bl.reshape(1, H), st, Wl2, Wr2)


def _tc_final(S2, C, bl2, st):
    def body(S_ref, C_ref, bl_ref, st_ref, o_ref):
        inv = 1.0 / jnp.maximum(C_ref[...][:, 0:1], 1.0)
        o_ref[...] = S_ref[...] * inv + bl_ref[...] + st_ref[...]

    return pl.pallas_call(
        body,
        grid=(N // _ROWS,),
        in_specs=[
            pl.BlockSpec((_ROWS, H), lambda i: (i, 0)),
            pl.BlockSpec((_ROWS, 16), lambda i: (i, 0)),
            pl.BlockSpec((1, H), lambda i: (0, 0)),
            pl.BlockSpec((_ROWS, H), lambda i: (i, 0)),
        ],
        out_specs=pl.BlockSpec((_ROWS, H), lambda i: (i, 0)),
        out_shape=jax.ShapeDtypeStruct((N, H), jnp.float32),
    )(S2, C, bl2.reshape(1, H), st)


def kernel(x_user, x_recipe, node_id_user, node_id_recipe, edge_index_u2r,
           edge_index_r2u, edge_label_index, W_user_lin, b_user_lin,
           W_recipe_lin, b_recipe_lin, emb_user, emb_recipe,
           Wl1_u2r, bl1_u2r, Wr1_u2r, Wl1_r2u, bl1_r2u, Wr1_r2u,
           Wl2_u2r, bl2_u2r, Wr2_u2r, Wl2_r2u, bl2_r2u, Wr2_r2u):
    xup = jnp.pad(x_user, ((0, 0), (0, 6)))
    Wup = jnp.pad(W_user_lin, ((0, 6), (0, 0)))

    pu1, su1 = _tc_encode(xup, emb_user, Wup, b_user_lin, Wl1_u2r, Wr1_r2u)
    pr1, sr1 = _tc_encode(x_recipe, emb_recipe, W_recipe_lin, b_recipe_lin,
                          Wl1_r2u, Wr1_u2r)

    msgs1 = jnp.concatenate([pr1, pu1], axis=0)
    src_all = jnp.concatenate(
        [edge_index_r2u[0], edge_index_u2r[0]]).reshape(NTILE, NCHUNK, CW)
    dst_all = jnp.concatenate(
        [edge_index_r2u[1], edge_index_u2r[1]]).reshape(NTILE, NCHUNK, CW)

    S1a, S1b, C = _sc_aggregate(msgs1[:, :HH], msgs1[:, HH:],
                                src_all, dst_all, with_counts=True)
    S1 = jnp.concatenate(
        [S1a.reshape(2, NP, HH), S1b.reshape(2, NP, HH)], axis=-1)
    C = C.reshape(2, NP, 16)
    Cu, Cr = C[0, :N], C[1, :N]

    pu2, su2 = _tc_layer_mid(S1[0, :N], Cu, bl1_r2u, su1, Wl2_u2r, Wr2_r2u)
    pr2, sr2 = _tc_layer_mid(S1[1, :N], Cr, bl1_u2r, sr1, Wl2_r2u, Wr2_u2r)

    msgs2 = jnp.concatenate([pr2, pu2], axis=0)
    S2a, S2b = _sc_aggregate(msgs2[:, :HH], msgs2[:, HH:],
                             src_all, dst_all, with_counts=False)
    S2 = jnp.concatenate(
        [S2a.reshape(2, NP, HH), S2b.reshape(2, NP, HH)], axis=-1)

    hu2 = _tc_final(S2[0, :N], Cu, bl2_r2u, su2)
    hr2 = _tc_final(S2[1, :N], Cr, bl2_u2r, sr2)

    pad = jnp.zeros((ELP - edge_label_index.shape[1],), jnp.int32)
    el0 = jnp.concatenate([edge_label_index[0], pad]).reshape(NTILE, SCH, SCW)
    el1 = jnp.concatenate([edge_label_index[1], pad]).reshape(NTILE, SCH, SCW)
    scores = _sc_score(hu2, hr2, el0, el1)
    return scores.reshape(-1)[:edge_label_index.shape[1]]


# speedup vs baseline: 6.8994x; 1.2643x over previous
import functools

import jax
import jax.numpy as jnp
from jax import lax
from jax.experimental import pallas as pl
from jax.experimental.pallas import tpu as pltpu
from jax.experimental.pallas import tpu_sc as plsc

N = 10000
NP = 10240
H = 128
HH = H // 2
E = 320000
NTILE = 32
EPT = E // 16
CW = 80
NCHUNK = EPT // CW
NB = 5
RPS = NP // 16

ELP = 102400
SCW = 64
SCH = ELP // (32 * SCW)


def _sc_mesh():
    return plsc.VectorSubcoreMesh(core_axis_name="c", subcore_axis_name="s")


def _zero_rows(ref, nrows, ncolchunks):
    zf = jnp.zeros((16,), jnp.float32)

    def body(i, _):
        for j in range(ncolchunks):
            ref[i, pl.ds(j * 16, 16)] = zf
        return 0

    lax.fori_loop(0, nrows, body, 0)


def _agg_body(with_counts, msgs0_hbm, msgs1_hbm, src_hbm, dst_hbm, *refs):
    if with_counts:
        (S0_out, S1_out, C_out, idx_src, idx_dst, zrow, acc,
         ones, zcnt, cnt) = refs[:10]
        rbs = refs[10:10 + NB]
        sems = refs[10 + NB:]
    else:
        (S0_out, S1_out, idx_src, idx_dst, zrow, acc) = refs[:6]
        rbs = refs[6:6 + NB]
        sems = refs[6 + NB:]

    c = lax.axis_index("c")
    s = lax.axis_index("s")
    w = c * 16 + s

    _zero_rows(zrow, 128, HH // 16)
    if with_counts:
        of = jnp.ones((16,), jnp.float32)

        def fill_ones(i, _):
            ones[i, :] = of
            return 0

        lax.fori_loop(0, CW, fill_ones, 0)
        _zero_rows(zcnt, 128, 1)

    pltpu.sync_copy(src_hbm.at[w], idx_src)
    pltpu.sync_copy(dst_hbm.at[w], idx_dst)

    offv = jnp.zeros((16,), jnp.int32) + c * N

    def offset_body(i, _):
        for j in range(CW // 16):
            sl = pl.ds(j * 16, 16)
            idx_src[i, sl] = idx_src[i, sl] + offv
        return 0

    lax.fori_loop(0, NCHUNK, offset_body, 0)

    for half in range(2):
        msgs_hbm = (msgs0_hbm, msgs1_hbm)[half]
        S_out = (S0_out, S1_out)[half]
        counts = with_counts and half == 0

        for k in range(RPS // 128):
            pltpu.sync_copy(zrow, acc.at[pl.ds(s * RPS + k * 128, 128)])
        if counts:
            for k in range(RPS // 128):
                pltpu.sync_copy(zcnt, cnt.at[pl.ds(s * RPS + k * 128, 128)])

        plsc.subcore_barrier()

        for b in range(NB):
            pltpu.async_copy(msgs_hbm.at[idx_src.at[b]], rbs[b], sems[b])

        def chunk_body(i, _):
            for b in range(NB):
                cch = i * NB + b
                pltpu.make_async_copy(
                    msgs_hbm.at[idx_src.at[cch]], rbs[b], sems[b]).wait()
                pltpu.sync_copy(rbs[b], acc.at[idx_dst.at[cch]], add=True)
                if counts:
                    pltpu.sync_copy(ones, cnt.at[idx_dst.at[cch]], add=True)

                @pl.when(cch + NB < NCHUNK)
                def _():
                    pltpu.async_copy(
                        msgs_hbm.at[idx_src.at[cch + NB]], rbs[b], sems[b])
            return 0

        lax.fori_loop(0, NCHUNK // NB, chunk_body, 0)

        plsc.subcore_barrier()

        pltpu.sync_copy(acc.at[pl.ds(s * RPS, RPS)], S_out.at[w])
        if counts:
            pltpu.sync_copy(cnt.at[pl.ds(s * RPS, RPS)], C_out.at[w])


def _sc_aggregate(msgs0, msgs1, src3, dst3, with_counts):
    out_type = [jax.ShapeDtypeStruct((NTILE, RPS, HH), jnp.float32)] * 2
    scratch = [
        pltpu.VMEM((NCHUNK, CW), jnp.int32),
        pltpu.VMEM((NCHUNK, CW), jnp.int32),
        pltpu.VMEM((128, HH), jnp.float32),
        pltpu.VMEM_SHARED((NP, HH), jnp.float32),
    ]
    if with_counts:
        out_type.append(jax.ShapeDtypeStruct((NTILE, RPS, 16), jnp.float32))
        scratch += [
            pltpu.VMEM((CW, 16), jnp.float32),
            pltpu.VMEM((128, 16), jnp.float32),
            pltpu.VMEM_SHARED((NP, 16), jnp.float32),
        ]
    scratch += [pltpu.VMEM((CW, HH), jnp.float32)] * NB
    scratch += [pltpu.SemaphoreType.DMA] * NB
    return pl.kernel(
        functools.partial(_agg_body, with_counts),
        out_type=out_type,
        mesh=_sc_mesh(),
        scratch_types=scratch,
        compiler_params=pltpu.CompilerParams(use_tc_tiling_on_sc=False),
    )(msgs0, msgs1, src3, dst3)


def _score_body(hu_hbm, hr_hbm, el0_hbm, el1_hbm, out_hbm,
                idx0, idx1, ub0, ub1, rb0, rb1, sc,
                usem0, usem1, rsem0, rsem1):
    c = lax.axis_index("c")
    s = lax.axis_index("s")
    w = c * 16 + s
    pltpu.sync_copy(el0_hbm.at[w], idx0)
    pltpu.sync_copy(el1_hbm.at[w], idx1)

    ubs, rbs = (ub0, ub1), (rb0, rb1)
    usems, rsems = (usem0, usem1), (rsem0, rsem1)
    for b in range(2):
        pltpu.async_copy(hu_hbm.at[idx0.at[b]], ubs[b], usems[b])
        pltpu.async_copy(hr_hbm.at[idx1.at[b]], rbs[b], rsems[b])

    lane = lax.iota(jnp.int32, 16)
    rows_g = [lane + g * 16 for g in range(SCW // 16)]

    def chunk(i, _):
        for b in range(2):
            cch = 2 * i + b
            ub, rb = ubs[b], rbs[b]
            pltpu.make_async_copy(hu_hbm.at[idx0.at[cch]], ub,
                                  usems[b]).wait()
            pltpu.make_async_copy(hr_hbm.at[idx1.at[cch]], rb,
                                  rsems[b]).wait()

            def hblk(ho, accs):
                for hh in range(8):
                    cols = (lane + (ho * 8 + hh)) & (H - 1)
                    new = []
                    for g in range(SCW // 16):
                        u = plsc.load_gather(ub, [rows_g[g], cols])
                        r = plsc.load_gather(rb, [rows_g[g], cols])
                        new.append(accs[g] + u * r)
                    accs = tuple(new)
                return accs

            accs = lax.fori_loop(
                0, H // 8, hblk,
                tuple(jnp.zeros((16,), jnp.float32)
                      for _ in range(SCW // 16)))
            for g in range(SCW // 16):
                sc[cch, pl.ds(g * 16, 16)] = accs[g]

            @pl.when(cch + 2 < SCH)
            def _():
                pltpu.async_copy(hu_hbm.at[idx0.at[cch + 2]], ub, usems[b])
                pltpu.async_copy(hr_hbm.at[idx1.at[cch + 2]], rb, rsems[b])
        return 0

    lax.fori_loop(0, SCH // 2, chunk, 0)
    pltpu.sync_copy(sc, out_hbm.at[w])


def _sc_score(hu2, hr2, el0, el1):
    return pl.kernel(
        _score_body,
        out_type=jax.ShapeDtypeStruct((NTILE, SCH, SCW), jnp.float32),
        mesh=_sc_mesh(),
        scratch_types=[
            pltpu.VMEM((SCH, SCW), jnp.int32),
            pltpu.VMEM((SCH, SCW), jnp.int32),
            pltpu.VMEM((SCW, H), jnp.float32),
            pltpu.VMEM((SCW, H), jnp.float32),
            pltpu.VMEM((SCW, H), jnp.float32),
            pltpu.VMEM((SCW, H), jnp.float32),
            pltpu.VMEM((SCH, SCW), jnp.float32),
            pltpu.SemaphoreType.DMA,
            pltpu.SemaphoreType.DMA,
            pltpu.SemaphoreType.DMA,
            pltpu.SemaphoreType.DMA,
        ],
        compiler_params=pltpu.CompilerParams(needs_layout_passes=False),
    )(hu2, hr2, el0, el1)


_ROWS = 1000


def _tc_encode(x, emb, W, b, Wl, Wr):
    n, k = x.shape

    def body(x_ref, emb_ref, W_ref, b_ref, Wl_ref, Wr_ref, p_ref, s_ref):
        h = jnp.dot(x_ref[...], W_ref[...], preferred_element_type=jnp.float32)
        h = h + b_ref[...] + emb_ref[...]
        p_ref[...] = jnp.dot(h, Wl_ref[...], preferred_element_type=jnp.float32)
        s_ref[...] = jnp.dot(h, Wr_ref[...], preferred_element_type=jnp.float32)

    return pl.pallas_call(
        body,
        grid=(n // _ROWS,),
        in_specs=[
            pl.BlockSpec((_ROWS, k), lambda i: (i, 0)),
            pl.BlockSpec((_ROWS, H), lambda i: (i, 0)),
            pl.BlockSpec((k, H), lambda i: (0, 0)),
            pl.BlockSpec((1, H), lambda i: (0, 0)),
            pl.BlockSpec((H, H), lambda i: (0, 0)),
            pl.BlockSpec((H, H), lambda i: (0, 0)),
        ],
        out_specs=[pl.BlockSpec((_ROWS, H), lambda i: (i, 0))] * 2,
        out_shape=[jax.ShapeDtypeStruct((n, H), jnp.float32)] * 2,
    )(x, emb, W, b.reshape(1, H), Wl, Wr)


def _tc_layer_mid(S, C, bl, st, Wl2, Wr2):
    def body(S_ref, C_ref, bl_ref, st_ref, Wl_ref, Wr_ref, p_ref, s_ref):
        inv = 1.0 / jnp.maximum(C_ref[...][:, 0:1], 1.0)
        h = jnp.maximum(S_ref[...] * inv + bl_ref[...] + st_ref[...], 0.0)
        p_ref[...] = jnp.dot(h, Wl_ref[...], preferred_element_type=jnp.float32)
        s_ref[...] = jnp.dot(h, Wr_ref[...], preferred_element_type=jnp.float32)

    return pl.pallas_call(
        body,
        grid=(N // _ROWS,),
        in_specs=[
            pl.BlockSpec((_ROWS, H), lambda i: (i, 0)),
            pl.BlockSpec((_ROWS, 16), lambda i: (i, 0)),
            pl.BlockSpec((1, H), lambda i: (0, 0)),
            pl.BlockSpec((_ROWS, H), lambda i: (i, 0)),
            pl.BlockSpec((H, H), lambda i: (0, 0)),
            pl.BlockSpec((H, H), lambda i: (0, 0)),
        ],
        out_specs=[pl.BlockSpec((_ROWS, H), lambda i: (i, 0))] * 2,
        out_shape=[jax.ShapeDtypeStruct((N, H), jnp.float32)] * 2,
    )(S, C, bl.reshape(1, H), st, Wl2, Wr2)


def _tc_final(S2, C, bl2, st):
    def body(S_ref, C_ref, bl_ref, st_ref, o_ref):
        inv = 1.0 / jnp.maximum(C_ref[...][:, 0:1], 1.0)
        o_ref[...] = S_ref[...] * inv + bl_ref[...] + st_ref[...]

    return pl.pallas_call(
        body,
        grid=(N // _ROWS,),
        in_specs=[
            pl.BlockSpec((_ROWS, H), lambda i: (i, 0)),
            pl.BlockSpec((_ROWS, 16), lambda i: (i, 0)),
            pl.BlockSpec((1, H), lambda i: (0, 0)),
            pl.BlockSpec((_ROWS, H), lambda i: (i, 0)),
        ],
        out_specs=pl.BlockSpec((_ROWS, H), lambda i: (i, 0)),
        out_shape=jax.ShapeDtypeStruct((N, H), jnp.float32),
    )(S2, C, bl2.reshape(1, H), st)


def kernel(x_user, x_recipe, node_id_user, node_id_recipe, edge_index_u2r,
           edge_index_r2u, edge_label_index, W_user_lin, b_user_lin,
           W_recipe_lin, b_recipe_lin, emb_user, emb_recipe,
           Wl1_u2r, bl1_u2r, Wr1_u2r, Wl1_r2u, bl1_r2u, Wr1_r2u,
           Wl2_u2r, bl2_u2r, Wr2_u2r, Wl2_r2u, bl2_r2u, Wr2_r2u):
    xup = jnp.pad(x_user, ((0, 0), (0, 6)))
    Wup = jnp.pad(W_user_lin, ((0, 6), (0, 0)))

    pu1, su1 = _tc_encode(xup, emb_user, Wup, b_user_lin, Wl1_u2r, Wr1_r2u)
    pr1, sr1 = _tc_encode(x_recipe, emb_recipe, W_recipe_lin, b_recipe_lin,
                          Wl1_r2u, Wr1_u2r)

    msgs1 = jnp.concatenate([pr1, pu1], axis=0)
    src_all = jnp.concatenate(
        [edge_index_r2u[0], edge_index_u2r[0]]).reshape(NTILE, NCHUNK, CW)
    dst_all = jnp.concatenate(
        [edge_index_r2u[1], edge_index_u2r[1]]).reshape(NTILE, NCHUNK, CW)

    S1a, S1b, C = _sc_aggregate(msgs1[:, :HH], msgs1[:, HH:],
                                src_all, dst_all, with_counts=True)
    S1 = jnp.concatenate(
        [S1a.reshape(2, NP, HH), S1b.reshape(2, NP, HH)], axis=-1)
    C = C.reshape(2, NP, 16)
    Cu, Cr = C[0, :N], C[1, :N]

    pu2, su2 = _tc_layer_mid(S1[0, :N], Cu, bl1_r2u, su1, Wl2_u2r, Wr2_r2u)
    pr2, sr2 = _tc_layer_mid(S1[1, :N], Cr, bl1_u2r, sr1, Wl2_r2u, Wr2_u2r)

    msgs2 = jnp.concatenate([pr2, pu2], axis=0)
    S2a, S2b = _sc_aggregate(msgs2[:, :HH], msgs2[:, HH:],
                             src_all, dst_all, with_counts=False)
    S2 = jnp.concatenate(
        [S2a.reshape(2, NP, HH), S2b.reshape(2, NP, HH)], axis=-1)

    hu2 = _tc_final(S2[0, :N], Cu, bl2_r2u, su2)
    hr2 = _tc_final(S2[1, :N], Cr, bl2_u2r, sr2)

    pad = jnp.zeros((ELP - edge_label_index.shape[1],), jnp.int32)
    el0 = jnp.concatenate([edge_label_index[0], pad]).reshape(NTILE, SCH, SCW)
    el1 = jnp.concatenate([edge_label_index[1], pad]).reshape(NTILE, SCH, SCW)
    scores = _sc_score(hu2, hr2, el0, el1)
    return scores.reshape(-1)[:edge_label_index.shape[1]]


# speedup vs baseline: 7.5561x; 1.0952x over previous
import functools

import jax
import jax.numpy as jnp
from jax import lax
from jax.experimental import pallas as pl
from jax.experimental.pallas import tpu as pltpu
from jax.experimental.pallas import tpu_sc as plsc

N = 10000
H = 128
HH = H // 2
E = 320000
NTILE = 32
EPT = E // 16
CW = 80
NCHUNK = EPT // CW
NB = 5
RPS = N // 16

ELP = 102400
SCW = 64
SCH = ELP // (32 * SCW)


def _sc_mesh():
    return plsc.VectorSubcoreMesh(core_axis_name="c", subcore_axis_name="s")


def _zero_rows(ref, nrows, ncolchunks):
    zf = jnp.zeros((16,), jnp.float32)

    def body(i, _):
        for j in range(ncolchunks):
            ref[i, pl.ds(j * 16, 16)] = zf
        return 0

    lax.fori_loop(0, nrows, body, 0)


def _agg_body(with_counts, msgs0_hbm, msgs1_hbm, src_hbm, dst_hbm, *refs):
    if with_counts:
        (S0_out, S1_out, C_out, idx_src, idx_dst, zrow, acc,
         ones, zcnt, cnt) = refs[:10]
        rbs = refs[10:10 + NB]
        sems = refs[10 + NB:]
    else:
        (S0_out, S1_out, idx_src, idx_dst, zrow, acc) = refs[:6]
        rbs = refs[6:6 + NB]
        sems = refs[6 + NB:]

    c = lax.axis_index("c")
    s = lax.axis_index("s")
    w = c * 16 + s

    _zero_rows(zrow, 125, HH // 16)
    if with_counts:
        of = jnp.ones((16,), jnp.float32)

        def fill_ones(i, _):
            ones[i, :] = of
            return 0

        lax.fori_loop(0, CW, fill_ones, 0)
        _zero_rows(zcnt, 125, 1)

    pltpu.sync_copy(src_hbm.at[w], idx_src)
    pltpu.sync_copy(dst_hbm.at[w], idx_dst)

    offv = jnp.zeros((16,), jnp.int32) + c * N

    def offset_body(i, _):
        for j in range(CW // 16):
            sl = pl.ds(j * 16, 16)
            idx_src[i, sl] = idx_src[i, sl] + offv
        return 0

    lax.fori_loop(0, NCHUNK, offset_body, 0)

    for half in range(2):
        msgs_hbm = (msgs0_hbm, msgs1_hbm)[half]
        S_out = (S0_out, S1_out)[half]
        counts = with_counts and half == 0

        for k in range(RPS // 125):
            pltpu.sync_copy(zrow, acc.at[pl.ds(s * RPS + k * 125, 125)])
        if counts:
            for k in range(RPS // 125):
                pltpu.sync_copy(zcnt, cnt.at[pl.ds(s * RPS + k * 125, 125)])

        plsc.subcore_barrier()

        for b in range(NB):
            pltpu.async_copy(msgs_hbm.at[idx_src.at[b]], rbs[b], sems[b])

        def chunk_body(i, _):
            for b in range(NB):
                cch = i * NB + b
                pltpu.make_async_copy(
                    msgs_hbm.at[idx_src.at[cch]], rbs[b], sems[b]).wait()
                pltpu.sync_copy(rbs[b], acc.at[idx_dst.at[cch]], add=True)
                if counts:
                    pltpu.sync_copy(ones, cnt.at[idx_dst.at[cch]], add=True)

                @pl.when(cch + NB < NCHUNK)
                def _():
                    pltpu.async_copy(
                        msgs_hbm.at[idx_src.at[cch + NB]], rbs[b], sems[b])
            return 0

        lax.fori_loop(0, NCHUNK // NB, chunk_body, 0)

        plsc.subcore_barrier()

        pltpu.sync_copy(acc.at[pl.ds(s * RPS, RPS)], S_out.at[w])
        if counts:
            pltpu.sync_copy(cnt.at[pl.ds(s * RPS, RPS)], C_out.at[w])


def _sc_aggregate(msgs0, msgs1, src3, dst3, with_counts):
    out_type = [jax.ShapeDtypeStruct((NTILE, RPS, HH), jnp.float32)] * 2
    scratch = [
        pltpu.VMEM((NCHUNK, CW), jnp.int32),
        pltpu.VMEM((NCHUNK, CW), jnp.int32),
        pltpu.VMEM((125, HH), jnp.float32),
        pltpu.VMEM_SHARED((N, HH), jnp.float32),
    ]
    if with_counts:
        out_type.append(jax.ShapeDtypeStruct((NTILE, RPS, 16), jnp.float32))
        scratch += [
            pltpu.VMEM((CW, 16), jnp.float32),
            pltpu.VMEM((125, 16), jnp.float32),
            pltpu.VMEM_SHARED((N, 16), jnp.float32),
        ]
    scratch += [pltpu.VMEM((CW, HH), jnp.float32)] * NB
    scratch += [pltpu.SemaphoreType.DMA] * NB
    return pl.kernel(
        functools.partial(_agg_body, with_counts),
        out_type=out_type,
        mesh=_sc_mesh(),
        scratch_types=scratch,
        compiler_params=pltpu.CompilerParams(use_tc_tiling_on_sc=False),
    )(msgs0, msgs1, src3, dst3)


def _score_body(h2_hbm, el0_hbm, el1_hbm, out_hbm,
                idx0, idx1, ub0, ub1, rb0, rb1, sc,
                usem0, usem1, rsem0, rsem1):
    c = lax.axis_index("c")
    s = lax.axis_index("s")
    w = c * 16 + s
    pltpu.sync_copy(el0_hbm.at[w], idx0)
    pltpu.sync_copy(el1_hbm.at[w], idx1)

    offv = jnp.zeros((16,), jnp.int32) + N

    def off_body(i, _):
        for j in range(SCW // 16):
            sl = pl.ds(j * 16, 16)
            idx0[i, sl] = idx0[i, sl] + offv
        return 0

    lax.fori_loop(0, SCH, off_body, 0)

    ubs, rbs = (ub0, ub1), (rb0, rb1)
    usems, rsems = (usem0, usem1), (rsem0, rsem1)
    for b in range(2):
        pltpu.async_copy(h2_hbm.at[idx0.at[b]], ubs[b], usems[b])
        pltpu.async_copy(h2_hbm.at[idx1.at[b]], rbs[b], rsems[b])

    lane = lax.iota(jnp.int32, 16)
    rows_g = [lane + g * 16 for g in range(SCW // 16)]

    def chunk(i, _):
        for b in range(2):
            cch = 2 * i + b
            ub, rb = ubs[b], rbs[b]
            pltpu.make_async_copy(h2_hbm.at[idx0.at[cch]], ub,
                                  usems[b]).wait()
            pltpu.make_async_copy(h2_hbm.at[idx1.at[cch]], rb,
                                  rsems[b]).wait()

            def hblk(ho, accs):
                for hh in range(8):
                    cols = (lane + (ho * 8 + hh)) & (H - 1)
                    new = []
                    for g in range(SCW // 16):
                        u = plsc.load_gather(ub, [rows_g[g], cols])
                        r = plsc.load_gather(rb, [rows_g[g], cols])
                        new.append(accs[g] + u * r)
                    accs = tuple(new)
                return accs

            accs = lax.fori_loop(
                0, H // 8, hblk,
                tuple(jnp.zeros((16,), jnp.float32)
                      for _ in range(SCW // 16)))
            for g in range(SCW // 16):
                sc[cch, pl.ds(g * 16, 16)] = accs[g]

            @pl.when(cch + 2 < SCH)
            def _():
                pltpu.async_copy(h2_hbm.at[idx0.at[cch + 2]], ub, usems[b])
                pltpu.async_copy(h2_hbm.at[idx1.at[cch + 2]], rb, rsems[b])
        return 0

    lax.fori_loop(0, SCH // 2, chunk, 0)
    pltpu.sync_copy(sc, out_hbm.at[w])


def _sc_score(h2, el0, el1):
    return pl.kernel(
        _score_body,
        out_type=jax.ShapeDtypeStruct((NTILE, SCH, SCW), jnp.float32),
        mesh=_sc_mesh(),
        scratch_types=[
            pltpu.VMEM((SCH, SCW), jnp.int32),
            pltpu.VMEM((SCH, SCW), jnp.int32),
            pltpu.VMEM((SCW, H), jnp.float32),
            pltpu.VMEM((SCW, H), jnp.float32),
            pltpu.VMEM((SCW, H), jnp.float32),
            pltpu.VMEM((SCW, H), jnp.float32),
            pltpu.VMEM((SCH, SCW), jnp.float32),
            pltpu.SemaphoreType.DMA,
            pltpu.SemaphoreType.DMA,
            pltpu.SemaphoreType.DMA,
            pltpu.SemaphoreType.DMA,
        ],
        compiler_params=pltpu.CompilerParams(needs_layout_passes=False),
    )(h2, el0, el1)


_ROWS = 1000
_G = N // _ROWS


def _half_idx(p):
    return jnp.where(p < _G, p + _G, p - _G)


def _tc_encode(xr, xu, emb_r, emb_u, W_r, W_u, b2, Wl2s, Wr2s):
    def body(xr_ref, xu_ref, er_ref, eu_ref, Wr_ref, Wu_ref, b_ref,
             Wl_ref, Wr2_ref, m0_ref, m1_ref, st_ref):
        p = pl.program_id(0)

        def emit(h):
            pm = jnp.dot(h, Wl_ref[0], preferred_element_type=jnp.float32)
            m0_ref[...] = pm[:, :HH]
            m1_ref[...] = pm[:, HH:]
            st_ref[...] = jnp.dot(h, Wr2_ref[0],
                                  preferred_element_type=jnp.float32)

        @pl.when(p < _G)
        def _():
            h = jnp.dot(xr_ref[...], Wr_ref[...],
                        preferred_element_type=jnp.float32)
            emit(h + b_ref[0] + er_ref[...])

        @pl.when(p >= _G)
        def _():
            h = jnp.dot(xu_ref[...], Wu_ref[...],
                        preferred_element_type=jnp.float32)
            emit(h + b_ref[0] + eu_ref[...])

    kr = xr.shape[1]
    ku = xu.shape[1]
    return pl.pallas_call(
        body,
        grid=(2 * _G,),
        in_specs=[
            pl.BlockSpec((_ROWS, kr), lambda p: (jnp.minimum(p, _G - 1), 0)),
            pl.BlockSpec((_ROWS, ku),
                         lambda p: (jnp.maximum(p, _G) - _G, 0)),
            pl.BlockSpec((_ROWS, H), lambda p: (jnp.minimum(p, _G - 1), 0)),
            pl.BlockSpec((_ROWS, H),
                         lambda p: (jnp.maximum(p, _G) - _G, 0)),
            pl.BlockSpec((kr, H), lambda p: (0, 0)),
            pl.BlockSpec((ku, H), lambda p: (0, 0)),
            pl.BlockSpec((1, 1, H), lambda p: (p // _G, 0, 0)),
            pl.BlockSpec((1, H, H), lambda p: (p // _G, 0, 0)),
            pl.BlockSpec((1, H, H), lambda p: (p // _G, 0, 0)),
        ],
        out_specs=[
            pl.BlockSpec((_ROWS, HH), lambda p: (p, 0)),
            pl.BlockSpec((_ROWS, HH), lambda p: (p, 0)),
            pl.BlockSpec((_ROWS, H), lambda p: (p, 0)),
        ],
        out_shape=[
            jax.ShapeDtypeStruct((2 * N, HH), jnp.float32),
            jax.ShapeDtypeStruct((2 * N, HH), jnp.float32),
            jax.ShapeDtypeStruct((2 * N, H), jnp.float32),
        ],
    )(xr, xu, emb_r, emb_u, W_r, W_u, b2, Wl2s, Wr2s)


def _tc_mid(Sa, Sb, Cf, st1, bl2, Wl2s, Wr2s):
    def body(Sa_ref, Sb_ref, C_ref, st_ref, b_ref, Wl_ref, Wr_ref,
             m0_ref, m1_ref, st2_ref):
        inv = 1.0 / jnp.maximum(C_ref[...][:, 0:1], 1.0)
        st = st_ref[...]
        b = b_ref[0]
        h0 = jnp.maximum(Sa_ref[...] * inv + b[:, :HH] + st[:, :HH], 0.0)
        h1 = jnp.maximum(Sb_ref[...] * inv + b[:, HH:] + st[:, HH:], 0.0)
        Wl = Wl_ref[0]
        Wr = Wr_ref[0]
        pm = (jnp.dot(h0, Wl[:HH], preferred_element_type=jnp.float32)
              + jnp.dot(h1, Wl[HH:], preferred_element_type=jnp.float32))
        m0_ref[...] = pm[:, :HH]
        m1_ref[...] = pm[:, HH:]
        st2_ref[...] = (
            jnp.dot(h0, Wr[:HH], preferred_element_type=jnp.float32)
            + jnp.dot(h1, Wr[HH:], preferred_element_type=jnp.float32))

    return pl.pallas_call(
        body,
        grid=(2 * _G,),
        in_specs=[
            pl.BlockSpec((_ROWS, HH), lambda p: (_half_idx(p), 0)),
            pl.BlockSpec((_ROWS, HH), lambda p: (_half_idx(p), 0)),
            pl.BlockSpec((_ROWS, 16), lambda p: (_half_idx(p), 0)),
            pl.BlockSpec((_ROWS, H), lambda p: (p, 0)),
            pl.BlockSpec((1, 1, H), lambda p: (p // _G, 0, 0)),
            pl.BlockSpec((1, H, H), lambda p: (p // _G, 0, 0)),
            pl.BlockSpec((1, H, H), lambda p: (p // _G, 0, 0)),
        ],
        out_specs=[
            pl.BlockSpec((_ROWS, HH), lambda p: (p, 0)),
            pl.BlockSpec((_ROWS, HH), lambda p: (p, 0)),
            pl.BlockSpec((_ROWS, H), lambda p: (p, 0)),
        ],
        out_shape=[
            jax.ShapeDtypeStruct((2 * N, HH), jnp.float32),
            jax.ShapeDtypeStruct((2 * N, HH), jnp.float32),
            jax.ShapeDtypeStruct((2 * N, H), jnp.float32),
        ],
    )(Sa, Sb, Cf, st1, bl2, Wl2s, Wr2s)


def _tc_final(Sa, Sb, Cf, st2, bl2):
    def body(Sa_ref, Sb_ref, C_ref, st_ref, b_ref, h2_ref):
        inv = 1.0 / jnp.maximum(C_ref[...][:, 0:1], 1.0)
        st = st_ref[...]
        b = b_ref[0]
        h0 = Sa_ref[...] * inv + b[:, :HH] + st[:, :HH]
        h1 = Sb_ref[...] * inv + b[:, HH:] + st[:, HH:]
        h2_ref[...] = jnp.concatenate([h0, h1], axis=-1)

    return pl.pallas_call(
        body,
        grid=(2 * _G,),
        in_specs=[
            pl.BlockSpec((_ROWS, HH), lambda p: (_half_idx(p), 0)),
            pl.BlockSpec((_ROWS, HH), lambda p: (_half_idx(p), 0)),
            pl.BlockSpec((_ROWS, 16), lambda p: (_half_idx(p), 0)),
            pl.BlockSpec((_ROWS, H), lambda p: (p, 0)),
            pl.BlockSpec((1, 1, H), lambda p: (p // _G, 0, 0)),
        ],
        out_specs=pl.BlockSpec((_ROWS, H), lambda p: (p, 0)),
        out_shape=jax.ShapeDtypeStruct((2 * N, H), jnp.float32),
    )(Sa, Sb, Cf, st2, bl2)


def kernel(x_user, x_recipe, node_id_user, node_id_recipe, edge_index_u2r,
           edge_index_r2u, edge_label_index, W_user_lin, b_user_lin,
           W_recipe_lin, b_recipe_lin, emb_user, emb_recipe,
           Wl1_u2r, bl1_u2r, Wr1_u2r, Wl1_r2u, bl1_r2u, Wr1_r2u,
           Wl2_u2r, bl2_u2r, Wr2_u2r, Wl2_r2u, bl2_r2u, Wr2_r2u):
    xup = jnp.pad(x_user, ((0, 0), (0, 6)))
    Wup = jnp.pad(W_user_lin, ((0, 6), (0, 0)))

    b2 = jnp.stack([b_recipe_lin, b_user_lin]).reshape(2, 1, H)
    Wl1s = jnp.stack([Wl1_r2u, Wl1_u2r])
    Wr1s = jnp.stack([Wr1_u2r, Wr1_r2u])
    bl1s = jnp.stack([bl1_u2r, bl1_r2u]).reshape(2, 1, H)
    Wl2s = jnp.stack([Wl2_r2u, Wl2_u2r])
    Wr2s = jnp.stack([Wr2_u2r, Wr2_r2u])
    bl2s = jnp.stack([bl2_u2r, bl2_r2u]).reshape(2, 1, H)

    m10, m11, st1 = _tc_encode(x_recipe, xup, emb_recipe, emb_user,
                               W_recipe_lin, Wup, b2, Wl1s, Wr1s)

    src_all = jnp.concatenate(
        [edge_index_r2u[0], edge_index_u2r[0]]).reshape(NTILE, NCHUNK, CW)
    dst_all = jnp.concatenate(
        [edge_index_r2u[1], edge_index_u2r[1]]).reshape(NTILE, NCHUNK, CW)

    S1a, S1b, C = _sc_aggregate(m10, m11, src_all, dst_all, with_counts=True)
    S1a = S1a.reshape(2 * N, HH)
    S1b = S1b.reshape(2 * N, HH)
    Cf = C.reshape(2 * N, 16)

    m20, m21, st2 = _tc_mid(S1a, S1b, Cf, st1, bl1s, Wl2s, Wr2s)

    S2a, S2b = _sc_aggregate(m20, m21, src_all, dst_all, with_counts=False)
    S2a = S2a.reshape(2 * N, HH)
    S2b = S2b.reshape(2 * N, HH)

    h2 = _tc_final(S2a, S2b, Cf, st2, bl2s)

    pad = jnp.zeros((ELP - edge_label_index.shape[1],), jnp.int32)
    el0 = jnp.concatenate([edge_label_index[0], pad]).reshape(NTILE, SCH, SCW)
    el1 = jnp.concatenate([edge_label_index[1], pad]).reshape(NTILE, SCH, SCW)
    scores = _sc_score(h2, el0, el1)
    return scores.reshape(-1)[:edge_label_index.shape[1]]


# speedup vs baseline: 9.6555x; 1.2778x over previous
import functools

import jax
import jax.numpy as jnp
from jax import lax
from jax.experimental import pallas as pl
from jax.experimental.pallas import tpu as pltpu
from jax.experimental.pallas import tpu_sc as plsc

N = 10000
H = 128
HH = H // 2
E = 320000
NTILE = 32
EPT = E // 16
CW = 80
NCHUNK = EPT // CW
NB = 5
RPS = N // 16

ELP = 102400
SCW = 64
SCH = ELP // (32 * SCW)


def _sc_mesh():
    return plsc.VectorSubcoreMesh(core_axis_name="c", subcore_axis_name="s")


def _zero_rows(ref, nrows, ncolchunks):
    zf = jnp.zeros((16,), jnp.float32)

    def body(i, _):
        for j in range(ncolchunks):
            ref[i, pl.ds(j * 16, 16)] = zf
        return 0

    lax.fori_loop(0, nrows, body, 0)


def _agg_body(with_counts, msgs_hbm, src_hbm, dst_hbm, *refs):
    if with_counts:
        (S_out, C_out, idx_src, idx_dst, zrow, acc,
         ones, zcnt, cnt) = refs[:9]
        rbs = refs[9:9 + NB]
        sems = refs[9 + NB:]
    else:
        (S_out, idx_src, idx_dst, zrow, acc) = refs[:5]
        rbs = refs[5:5 + NB]
        sems = refs[5 + NB:]

    c = lax.axis_index("c")
    s = lax.axis_index("s")
    w = c * 16 + s

    zb = jnp.zeros((32,), jnp.bfloat16)

    def zrow_body(i, _):
        for j in range(H // 32):
            zrow[i, pl.ds(j * 32, 32)] = zb
        return 0

    lax.fori_loop(0, 125, zrow_body, 0)
    if with_counts:
        of = jnp.ones((16,), jnp.float32)

        def fill_ones(i, _):
            ones[i, :] = of
            return 0

        lax.fori_loop(0, CW, fill_ones, 0)
        _zero_rows(zcnt, 125, 1)

    pltpu.sync_copy(src_hbm.at[w], idx_src)
    pltpu.sync_copy(dst_hbm.at[w], idx_dst)

    offv = jnp.zeros((16,), jnp.int32) + c * N

    def offset_body(i, _):
        for j in range(CW // 16):
            sl = pl.ds(j * 16, 16)
            idx_src[i, sl] = idx_src[i, sl] + offv
        return 0

    lax.fori_loop(0, NCHUNK, offset_body, 0)

    for k in range(RPS // 125):
        pltpu.sync_copy(zrow, acc.at[pl.ds(s * RPS + k * 125, 125)])
    if with_counts:
        for k in range(RPS // 125):
            pltpu.sync_copy(zcnt, cnt.at[pl.ds(s * RPS + k * 125, 125)])

    plsc.subcore_barrier()

    for b in range(NB):
        pltpu.async_copy(msgs_hbm.at[idx_src.at[b]], rbs[b], sems[b])

    def chunk_body(i, _):
        for b in range(NB):
            cch = i * NB + b
            pltpu.make_async_copy(
                msgs_hbm.at[idx_src.at[cch]], rbs[b], sems[b]).wait()
            pltpu.sync_copy(rbs[b], acc.at[idx_dst.at[cch]], add=True)
            if with_counts:
                pltpu.sync_copy(ones, cnt.at[idx_dst.at[cch]], add=True)

            @pl.when(cch + NB < NCHUNK)
            def _():
                pltpu.async_copy(
                    msgs_hbm.at[idx_src.at[cch + NB]], rbs[b], sems[b])
        return 0

    lax.fori_loop(0, NCHUNK // NB, chunk_body, 0)

    plsc.subcore_barrier()

    pltpu.sync_copy(acc.at[pl.ds(s * RPS, RPS)], S_out.at[w])
    if with_counts:
        pltpu.sync_copy(cnt.at[pl.ds(s * RPS, RPS)], C_out.at[w])


def _sc_aggregate(msgs, src3, dst3, with_counts):
    out_type = [jax.ShapeDtypeStruct((NTILE, RPS, H), jnp.bfloat16)]
    scratch = [
        pltpu.VMEM((NCHUNK, CW), jnp.int32),
        pltpu.VMEM((NCHUNK, CW), jnp.int32),
        pltpu.VMEM((125, H), jnp.bfloat16),
        pltpu.VMEM_SHARED((N, H), jnp.bfloat16),
    ]
    if with_counts:
        out_type.append(jax.ShapeDtypeStruct((NTILE, RPS, 16), jnp.float32))
        scratch += [
            pltpu.VMEM((CW, 16), jnp.float32),
            pltpu.VMEM((125, 16), jnp.float32),
            pltpu.VMEM_SHARED((N, 16), jnp.float32),
        ]
    scratch += [pltpu.VMEM((CW, H), jnp.bfloat16)] * NB
    scratch += [pltpu.SemaphoreType.DMA] * NB
    return pl.kernel(
        functools.partial(_agg_body, with_counts),
        out_type=out_type,
        mesh=_sc_mesh(),
        scratch_types=scratch,
        compiler_params=pltpu.CompilerParams(use_tc_tiling_on_sc=False),
    )(msgs, src3, dst3)


def _score_body(h2_hbm, el0_hbm, el1_hbm, out_hbm,
                idx0, idx1, ub0, ub1, rb0, rb1, sc,
                usem0, usem1, rsem0, rsem1):
    c = lax.axis_index("c")
    s = lax.axis_index("s")
    w = c * 16 + s
    pltpu.sync_copy(el0_hbm.at[w], idx0)
    pltpu.sync_copy(el1_hbm.at[w], idx1)

    offv = jnp.zeros((16,), jnp.int32) + N

    def off_body(i, _):
        for j in range(SCW // 16):
            sl = pl.ds(j * 16, 16)
            idx0[i, sl] = idx0[i, sl] + offv
        return 0

    lax.fori_loop(0, SCH, off_body, 0)

    ubs, rbs = (ub0, ub1), (rb0, rb1)
    usems, rsems = (usem0, usem1), (rsem0, rsem1)
    for b in range(2):
        pltpu.async_copy(h2_hbm.at[idx0.at[b]], ubs[b], usems[b])
        pltpu.async_copy(h2_hbm.at[idx1.at[b]], rbs[b], rsems[b])

    lane = lax.iota(jnp.int32, 16)
    rows_g = [lane + g * 16 for g in range(SCW // 16)]

    def chunk(i, _):
        for b in range(2):
            cch = 2 * i + b
            ub, rb = ubs[b], rbs[b]
            pltpu.make_async_copy(h2_hbm.at[idx0.at[cch]], ub,
                                  usems[b]).wait()
            pltpu.make_async_copy(h2_hbm.at[idx1.at[cch]], rb,
                                  rsems[b]).wait()

            def hblk(ho, accs):
                for hh in range(8):
                    cols = (lane + (ho * 8 + hh)) & (H - 1)
                    new = []
                    for g in range(SCW // 16):
                        u = plsc.load_gather(ub, [rows_g[g], cols])
                        r = plsc.load_gather(rb, [rows_g[g], cols])
                        new.append(accs[g] + u * r)
                    accs = tuple(new)
                return accs

            accs = lax.fori_loop(
                0, H // 8, hblk,
                tuple(jnp.zeros((16,), jnp.float32)
                      for _ in range(SCW // 16)))
            for g in range(SCW // 16):
                sc[cch, pl.ds(g * 16, 16)] = accs[g]

            @pl.when(cch + 2 < SCH)
            def _():
                pltpu.async_copy(h2_hbm.at[idx0.at[cch + 2]], ub, usems[b])
                pltpu.async_copy(h2_hbm.at[idx1.at[cch + 2]], rb, rsems[b])
        return 0

    lax.fori_loop(0, SCH // 2, chunk, 0)
    pltpu.sync_copy(sc, out_hbm.at[w])


def _sc_score(h2, el0, el1):
    return pl.kernel(
        _score_body,
        out_type=jax.ShapeDtypeStruct((NTILE, SCH, SCW), jnp.float32),
        mesh=_sc_mesh(),
        scratch_types=[
            pltpu.VMEM((SCH, SCW), jnp.int32),
            pltpu.VMEM((SCH, SCW), jnp.int32),
            pltpu.VMEM((SCW, H), jnp.float32),
            pltpu.VMEM((SCW, H), jnp.float32),
            pltpu.VMEM((SCW, H), jnp.float32),
            pltpu.VMEM((SCW, H), jnp.float32),
            pltpu.VMEM((SCH, SCW), jnp.float32),
            pltpu.SemaphoreType.DMA,
            pltpu.SemaphoreType.DMA,
            pltpu.SemaphoreType.DMA,
            pltpu.SemaphoreType.DMA,
        ],
        compiler_params=pltpu.CompilerParams(needs_layout_passes=False),
    )(h2, el0, el1)


_ROWS = 1000
_G = N // _ROWS


def _half_idx(p):
    return jnp.where(p < _G, p + _G, p - _G)


def _tc_encode(xr, xu, emb_r, emb_u, W_r, W_u, b2, Wl2s, Wr2s):
    def body(xr_ref, xu_ref, er_ref, eu_ref, Wr_ref, Wu_ref, b_ref,
             Wl_ref, Wr2_ref, m_ref, st_ref):
        p = pl.program_id(0)

        def emit(h):
            pm = jnp.dot(h, Wl_ref[0], preferred_element_type=jnp.float32)
            m_ref[...] = pm.astype(jnp.bfloat16)
            st_ref[...] = jnp.dot(h, Wr2_ref[0],
                                  preferred_element_type=jnp.float32)

        @pl.when(p < _G)
        def _():
            h = jnp.dot(xr_ref[...], Wr_ref[...],
                        preferred_element_type=jnp.float32)
            emit(h + b_ref[0] + er_ref[...])

        @pl.when(p >= _G)
        def _():
            h = jnp.dot(xu_ref[...], Wu_ref[...],
                        preferred_element_type=jnp.float32)
            emit(h + b_ref[0] + eu_ref[...])

    kr = xr.shape[1]
    ku = xu.shape[1]
    return pl.pallas_call(
        body,
        grid=(2 * _G,),
        in_specs=[
            pl.BlockSpec((_ROWS, kr), lambda p: (jnp.minimum(p, _G - 1), 0)),
            pl.BlockSpec((_ROWS, ku),
                         lambda p: (jnp.maximum(p, _G) - _G, 0)),
            pl.BlockSpec((_ROWS, H), lambda p: (jnp.minimum(p, _G - 1), 0)),
            pl.BlockSpec((_ROWS, H),
                         lambda p: (jnp.maximum(p, _G) - _G, 0)),
            pl.BlockSpec((kr, H), lambda p: (0, 0)),
            pl.BlockSpec((ku, H), lambda p: (0, 0)),
            pl.BlockSpec((1, 1, H), lambda p: (p // _G, 0, 0)),
            pl.BlockSpec((1, H, H), lambda p: (p // _G, 0, 0)),
            pl.BlockSpec((1, H, H), lambda p: (p // _G, 0, 0)),
        ],
        out_specs=[
            pl.BlockSpec((_ROWS, H), lambda p: (p, 0)),
            pl.BlockSpec((_ROWS, H), lambda p: (p, 0)),
        ],
        out_shape=[
            jax.ShapeDtypeStruct((2 * N, H), jnp.bfloat16),
            jax.ShapeDtypeStruct((2 * N, H), jnp.float32),
        ],
    )(xr, xu, emb_r, emb_u, W_r, W_u, b2, Wl2s, Wr2s)


def _tc_mid(S1, Cf, st1, bl2, Wl2s, Wr2s):
    def body(S_ref, C_ref, st_ref, b_ref, Wl_ref, Wr_ref,
             m_ref, st2_ref):
        inv = 1.0 / jnp.maximum(C_ref[...][:, 0:1], 1.0)
        h = jnp.maximum(
            S_ref[...].astype(jnp.float32) * inv + b_ref[0] + st_ref[...],
            0.0)
        pm = jnp.dot(h, Wl_ref[0], preferred_element_type=jnp.float32)
        m_ref[...] = pm.astype(jnp.bfloat16)
        st2_ref[...] = jnp.dot(h, Wr_ref[0],
                               preferred_element_type=jnp.float32)

    return pl.pallas_call(
        body,
        grid=(2 * _G,),
        in_specs=[
            pl.BlockSpec((_ROWS, H), lambda p: (_half_idx(p), 0)),
            pl.BlockSpec((_ROWS, 16), lambda p: (_half_idx(p), 0)),
            pl.BlockSpec((_ROWS, H), lambda p: (p, 0)),
            pl.BlockSpec((1, 1, H), lambda p: (p // _G, 0, 0)),
            pl.BlockSpec((1, H, H), lambda p: (p // _G, 0, 0)),
            pl.BlockSpec((1, H, H), lambda p: (p // _G, 0, 0)),
        ],
        out_specs=[
            pl.BlockSpec((_ROWS, H), lambda p: (p, 0)),
            pl.BlockSpec((_ROWS, H), lambda p: (p, 0)),
        ],
        out_shape=[
            jax.ShapeDtypeStruct((2 * N, H), jnp.bfloat16),
            jax.ShapeDtypeStruct((2 * N, H), jnp.float32),
        ],
    )(S1, Cf, st1, bl2, Wl2s, Wr2s)


def _tc_final(S2, Cf, st2, bl2):
    def body(S_ref, C_ref, st_ref, b_ref, h2_ref):
        inv = 1.0 / jnp.maximum(C_ref[...][:, 0:1], 1.0)
        h2_ref[...] = (S_ref[...].astype(jnp.float32) * inv
                       + b_ref[0] + st_ref[...])

    return pl.pallas_call(
        body,
        grid=(2 * _G,),
        in_specs=[
            pl.BlockSpec((_ROWS, H), lambda p: (_half_idx(p), 0)),
            pl.BlockSpec((_ROWS, 16), lambda p: (_half_idx(p), 0)),
            pl.BlockSpec((_ROWS, H), lambda p: (p, 0)),
            pl.BlockSpec((1, 1, H), lambda p: (p // _G, 0, 0)),
        ],
        out_specs=pl.BlockSpec((_ROWS, H), lambda p: (p, 0)),
        out_shape=jax.ShapeDtypeStruct((2 * N, H), jnp.float32),
    )(S2, Cf, st2, bl2)


def kernel(x_user, x_recipe, node_id_user, node_id_recipe, edge_index_u2r,
           edge_index_r2u, edge_label_index, W_user_lin, b_user_lin,
           W_recipe_lin, b_recipe_lin, emb_user, emb_recipe,
           Wl1_u2r, bl1_u2r, Wr1_u2r, Wl1_r2u, bl1_r2u, Wr1_r2u,
           Wl2_u2r, bl2_u2r, Wr2_u2r, Wl2_r2u, bl2_r2u, Wr2_r2u):
    xup = jnp.pad(x_user, ((0, 0), (0, 6)))
    Wup = jnp.pad(W_user_lin, ((0, 6), (0, 0)))

    b2 = jnp.stack([b_recipe_lin, b_user_lin]).reshape(2, 1, H)
    Wl1s = jnp.stack([Wl1_r2u, Wl1_u2r])
    Wr1s = jnp.stack([Wr1_u2r, Wr1_r2u])
    bl1s = jnp.stack([bl1_u2r, bl1_r2u]).reshape(2, 1, H)
    Wl2s = jnp.stack([Wl2_r2u, Wl2_u2r])
    Wr2s = jnp.stack([Wr2_u2r, Wr2_r2u])
    bl2s = jnp.stack([bl2_u2r, bl2_r2u]).reshape(2, 1, H)

    m1, st1 = _tc_encode(x_recipe, xup, emb_recipe, emb_user,
                         W_recipe_lin, Wup, b2, Wl1s, Wr1s)

    src_all = jnp.concatenate(
        [edge_index_r2u[0], edge_index_u2r[0]]).reshape(NTILE, NCHUNK, CW)
    dst_all = jnp.concatenate(
        [edge_index_r2u[1], edge_index_u2r[1]]).reshape(NTILE, NCHUNK, CW)

    S1, C = _sc_aggregate(m1, src_all, dst_all, with_counts=True)
    Cf = C.reshape(2 * N, 16)

    m2, st2 = _tc_mid(S1.reshape(2 * N, H), Cf, st1, bl1s, Wl2s, Wr2s)

    (S2,) = _sc_aggregate(m2, src_all, dst_all, with_counts=False)

    h2 = _tc_final(S2.reshape(2 * N, H), Cf, st2, bl2s)

    pad = jnp.zeros((ELP - edge_label_index.shape[1],), jnp.int32)
    el0 = jnp.concatenate([edge_label_index[0], pad]).reshape(NTILE, SCH, SCW)
    el1 = jnp.concatenate([edge_label_index[1], pad]).reshape(NTILE, SCH, SCW)
    scores = _sc_score(h2, el0, el1)
    return scores.reshape(-1)[:edge_label_index.shape[1]]
